# baseline (speedup 1.0000x reference)
import functools

import jax
import jax.numpy as jnp
from jax import lax
from jax.experimental import pallas as pl
from jax.experimental.pallas import tpu as pltpu
from jax.experimental.pallas import tpu_sc as plsc

B, J, V = 32, 8, 32768
SPLIT = 16384
SC_COLS = V - SPLIT
VC = 2048
NCHUNK_TC = SPLIT // VC
CH = 2048
NCH_SC = SC_COLS // CH
NGRP = CH // 16
NC, NS = 2, 16


def _tc_body(a_ref, e_ref, logits_ref, stddev_ref, bestv_ref, besti_ref,
             macc_ref, cidx_ref):
    j = pl.program_id(0)

    a = a_ref[...]
    e = e_ref[...]

    m = jnp.min(a, axis=1)
    iota_j = jax.lax.broadcasted_iota(jnp.int32, (B, J, VC), 1)
    jsel = jnp.min(jnp.where(a == m[:, None, :], iota_j, J), axis=1)
    s = jnp.sum(jnp.where(iota_j == jsel[:, None, :], e, 0.0), axis=1)
    logits_ref[...] = m
    stddev_ref[...] = s

    @pl.when(j == 0)
    def _():
        macc_ref[...] = a
        cidx_ref[...] = jnp.zeros((B, J, VC), jnp.int32)

    @pl.when(j > 0)
    def _():
        upd = a > macc_ref[...]
        macc_ref[...] = jnp.where(upd, a, macc_ref[...])
        cidx_ref[...] = jnp.where(upd, j, cidx_ref[...])

    @pl.when(j == NCHUNK_TC - 1)
    def _():
        macc = macc_ref[...]
        cm = jnp.max(macc, axis=2)
        lane = jax.lax.broadcasted_iota(jnp.int32, (B, J, VC), 2)
        gidx = cidx_ref[...] * VC + lane
        bestv_ref[...] = cm
        besti_ref[...] = jnp.min(
            jnp.where(macc == cm[:, :, None], gidx, V), axis=2)


def _tc_head(action, explore_rate):
    return pl.pallas_call(
        _tc_body,
        grid=(NCHUNK_TC,),
        in_specs=[
            pl.BlockSpec((B, J, VC), lambda j: (0, 0, j)),
            pl.BlockSpec((B, J, VC), lambda j: (0, 0, j)),
        ],
        out_specs=[
            pl.BlockSpec((B, VC), lambda j: (0, j)),
            pl.BlockSpec((B, VC), lambda j: (0, j)),
            pl.BlockSpec((B, J), lambda j: (0, 0)),
            pl.BlockSpec((B, J), lambda j: (0, 0)),
        ],
        out_shape=[
            jax.ShapeDtypeStruct((B, SPLIT), jnp.float32),
            jax.ShapeDtypeStruct((B, SPLIT), jnp.float32),
            jax.ShapeDtypeStruct((B, J), jnp.float32),
            jax.ShapeDtypeStruct((B, J), jnp.int32),
        ],
        scratch_shapes=[
            pltpu.VMEM((B, J, VC), jnp.float32),
            pltpu.VMEM((B, J, VC), jnp.int32),
        ],
        compiler_params=pltpu.CompilerParams(
            dimension_semantics=("arbitrary",),
        ),
    )(action, explore_rate)


def _sc_tail(action, explore_rate):
    mesh = plsc.VectorSubcoreMesh(core_axis_name="c", subcore_axis_name="s")

    @functools.partial(
        pl.kernel,
        out_type=[
            jax.ShapeDtypeStruct((B, SC_COLS), jnp.float32),
            jax.ShapeDtypeStruct((B, SC_COLS), jnp.float32),
            jax.ShapeDtypeStruct((B, J, 16), jnp.float32),
            jax.ShapeDtypeStruct((B, J, 16), jnp.int32),
        ],
        mesh=mesh,
        scratch_types=[
            pltpu.VMEM((J, CH), jnp.float32),
            pltpu.VMEM((J, CH), jnp.float32),
            pltpu.VMEM((CH,), jnp.float32),
            pltpu.VMEM((CH,), jnp.float32),
            pltpu.VMEM((J, 16), jnp.float32),
            pltpu.VMEM((J, 16), jnp.int32),
        ],
    )
    def sc_kernel(a_hbm, e_hbm, logits_hbm, stddev_hbm, mv_hbm, mi_hbm,
                  a_v, e_v, m_v, s_v, rm_v, ri_v):
        b = lax.axis_index("s") * NC + lax.axis_index("c")
        lane = jax.lax.broadcasted_iota(jnp.int32, (16,), 0)

        for jj in range(J):
            rm_v[jj, :] = jnp.full((16,), -jnp.inf, jnp.float32)
            ri_v[jj, :] = jnp.zeros((16,), jnp.int32)

        @pl.loop(0, NCH_SC)
        def _(c):
            off = SPLIT + c * CH
            pltpu.sync_copy(a_hbm.at[b, :, pl.ds(off, CH)], a_v)
            pltpu.sync_copy(e_hbm.at[b, :, pl.ds(off, CH)], e_v)

            rm0 = [rm_v[jj, :] for jj in range(J)]
            ri0 = [ri_v[jj, :] for jj in range(J)]

            def group(g, carry):
                rm, ri = carry
                col = g * 16
                idx0 = off + col + lane
                a0 = a_v[0, pl.ds(col, 16)]
                m = a0
                s = e_v[0, pl.ds(col, 16)]
                u0 = a0 > rm[0]
                new_rm = [jnp.where(u0, a0, rm[0])]
                new_ri = [jnp.where(u0, idx0, ri[0])]
                for jj in range(1, J):
                    aj = a_v[jj, pl.ds(col, 16)]
                    upd = aj < m
                    m = jnp.where(upd, aj, m)
                    s = jnp.where(upd, e_v[jj, pl.ds(col, 16)], s)
                    uj = aj > rm[jj]
                    new_rm.append(jnp.where(uj, aj, rm[jj]))
                    new_ri.append(jnp.where(uj, idx0, ri[jj]))
                m_v[pl.ds(col, 16)] = m
                s_v[pl.ds(col, 16)] = s
                return new_rm, new_ri

            rm, ri = lax.fori_loop(0, NGRP, group, (rm0, ri0))
            for jj in range(J):
                rm_v[jj, :] = rm[jj]
                ri_v[jj, :] = ri[jj]

            pltpu.sync_copy(m_v, logits_hbm.at[b, pl.ds(c * CH, CH)])
            pltpu.sync_copy(s_v, stddev_hbm.at[b, pl.ds(c * CH, CH)])

        pltpu.sync_copy(rm_v, mv_hbm.at[b])
        pltpu.sync_copy(ri_v, mi_hbm.at[b])

    return sc_kernel(action, explore_rate)


def _merge_body(tv_ref, ti_ref, sv_ref, si_ref, best_ref):
    sv = sv_ref[...]
    si = si_ref[...]
    scm = jnp.max(sv, axis=2)
    sci = jnp.min(jnp.where(sv == scm[:, :, None], si, V), axis=2)
    best_ref[...] = jnp.where(scm > tv_ref[...], sci, ti_ref[...])


def _merge(tv, ti, sv, si):
    return pl.pallas_call(
        _merge_body,
        out_shape=jax.ShapeDtypeStruct((B, J), jnp.int32),
    )(tv, ti, sv, si)


@jax.jit
def kernel(action, explore_rate):
    lg_tc, sd_tc, tv, ti = _tc_head(action, explore_rate)
    lg_sc, sd_sc, sv, si = _sc_tail(action, explore_rate)
    best2d = _merge(tv, ti, sv, si)
    logits = jnp.concatenate([lg_tc, lg_sc], axis=1)
    stddev = jnp.concatenate([sd_tc, sd_sc], axis=1)
    return logits, stddev, best2d[:, None, :]


# speedup vs baseline: 1.0020x; 1.0020x over previous
import functools

import jax
import jax.numpy as jnp
from jax import lax
from jax.experimental import pallas as pl
from jax.experimental.pallas import tpu as pltpu
from jax.experimental.pallas import tpu_sc as plsc

B, J, V = 32, 8, 32768
SPLIT = 16384
SC_COLS = V - SPLIT
VC = 2048
NCHUNK_TC = SPLIT // VC
CH = 2048
NCH_SC = SC_COLS // CH
NGRP = CH // 16
NC, NS = 2, 16


def _tc_body(a_ref, e_ref, logits_ref, stddev_ref, bestv_ref, besti_ref,
             macc_ref, cidx_ref):
    j = pl.program_id(0)

    a = a_ref[...]
    e = e_ref[...]

    m = jnp.min(a, axis=1)
    iota_j = jax.lax.broadcasted_iota(jnp.int32, (B, J, VC), 1)
    jsel = jnp.min(jnp.where(a == m[:, None, :], iota_j, J), axis=1)
    s = jnp.sum(jnp.where(iota_j == jsel[:, None, :], e, 0.0), axis=1)
    logits_ref[...] = m
    stddev_ref[...] = s

    @pl.when(j == 0)
    def _():
        macc_ref[...] = a
        cidx_ref[...] = jnp.zeros((B, J, VC), jnp.int32)

    @pl.when(j > 0)
    def _():
        upd = a > macc_ref[...]
        macc_ref[...] = jnp.where(upd, a, macc_ref[...])
        cidx_ref[...] = jnp.where(upd, j, cidx_ref[...])

    @pl.when(j == NCHUNK_TC - 1)
    def _():
        macc = macc_ref[...]
        cm = jnp.max(macc, axis=2)
        lane = jax.lax.broadcasted_iota(jnp.int32, (B, J, VC), 2)
        gidx = cidx_ref[...] * VC + lane
        bestv_ref[...] = cm
        besti_ref[...] = jnp.min(
            jnp.where(macc == cm[:, :, None], gidx, V), axis=2)


def _tc_head(action, explore_rate):
    return pl.pallas_call(
        _tc_body,
        grid=(NCHUNK_TC,),
        in_specs=[
            pl.BlockSpec((B, J, VC), lambda j: (0, 0, j)),
            pl.BlockSpec((B, J, VC), lambda j: (0, 0, j)),
        ],
        out_specs=[
            pl.BlockSpec((B, VC), lambda j: (0, j)),
            pl.BlockSpec((B, VC), lambda j: (0, j)),
            pl.BlockSpec((B, J), lambda j: (0, 0)),
            pl.BlockSpec((B, J), lambda j: (0, 0)),
        ],
        out_shape=[
            jax.ShapeDtypeStruct((B, SPLIT), jnp.float32),
            jax.ShapeDtypeStruct((B, SPLIT), jnp.float32),
            jax.ShapeDtypeStruct((B, J), jnp.float32),
            jax.ShapeDtypeStruct((B, J), jnp.int32),
        ],
        scratch_shapes=[
            pltpu.VMEM((B, J, VC), jnp.float32),
            pltpu.VMEM((B, J, VC), jnp.int32),
        ],
        compiler_params=pltpu.CompilerParams(
            dimension_semantics=("arbitrary",),
        ),
    )(action, explore_rate)


def _sc_tail(action, explore_rate):
    mesh = plsc.VectorSubcoreMesh(core_axis_name="c", subcore_axis_name="s")

    @functools.partial(
        pl.kernel,
        out_type=[
            jax.ShapeDtypeStruct((B, SC_COLS), jnp.float32),
            jax.ShapeDtypeStruct((B, SC_COLS), jnp.float32),
            jax.ShapeDtypeStruct((B, J, 16), jnp.float32),
            jax.ShapeDtypeStruct((B, J, 16), jnp.int32),
        ],
        mesh=mesh,
        scratch_types=[
            pltpu.VMEM((J, CH), jnp.float32),
            pltpu.VMEM((J, CH), jnp.float32),
            pltpu.VMEM((CH,), jnp.float32),
            pltpu.VMEM((CH,), jnp.float32),
            pltpu.VMEM((J, 16), jnp.float32),
            pltpu.VMEM((J, 16), jnp.int32),
        ],
    )
    def sc_kernel(a_hbm, e_hbm, logits_hbm, stddev_hbm, mv_hbm, mi_hbm,
                  a_v, e_v, m_v, s_v, rm_v, ri_v):
        b = lax.axis_index("s") * NC + lax.axis_index("c")
        lane = jax.lax.broadcasted_iota(jnp.int32, (16,), 0)

        for jj in range(J):
            rm_v[jj, :] = jnp.full((16,), -jnp.inf, jnp.float32)
            ri_v[jj, :] = jnp.zeros((16,), jnp.int32)

        @pl.loop(0, NCH_SC)
        def _(c):
            off = SPLIT + c * CH
            pltpu.sync_copy(a_hbm.at[b, :, pl.ds(off, CH)], a_v)
            pltpu.sync_copy(e_hbm.at[b, :, pl.ds(off, CH)], e_v)

            rm0 = [rm_v[jj, :] for jj in range(J)]
            ri0 = [ri_v[jj, :] for jj in range(J)]

            def group(g, carry):
                rm, ri = carry
                col = g * 16
                idx0 = off + col + lane
                a0 = a_v[0, pl.ds(col, 16)]
                m = a0
                s = e_v[0, pl.ds(col, 16)]
                u0 = a0 > rm[0]
                new_rm = [jnp.where(u0, a0, rm[0])]
                new_ri = [jnp.where(u0, idx0, ri[0])]
                for jj in range(1, J):
                    aj = a_v[jj, pl.ds(col, 16)]
                    upd = aj < m
                    m = jnp.where(upd, aj, m)
                    s = jnp.where(upd, e_v[jj, pl.ds(col, 16)], s)
                    uj = aj > rm[jj]
                    new_rm.append(jnp.where(uj, aj, rm[jj]))
                    new_ri.append(jnp.where(uj, idx0, ri[jj]))
                m_v[pl.ds(col, 16)] = m
                s_v[pl.ds(col, 16)] = s
                return new_rm, new_ri

            rm, ri = lax.fori_loop(0, NGRP, group, (rm0, ri0))
            for jj in range(J):
                rm_v[jj, :] = rm[jj]
                ri_v[jj, :] = ri[jj]

            pltpu.sync_copy(m_v, logits_hbm.at[b, pl.ds(c * CH, CH)])
            pltpu.sync_copy(s_v, stddev_hbm.at[b, pl.ds(c * CH, CH)])

        pltpu.sync_copy(rm_v, mv_hbm.at[b])
        pltpu.sync_copy(ri_v, mi_hbm.at[b])

    return sc_kernel(action, explore_rate)


def _merge_body(tv_ref, ti_ref, sv_ref, si_ref, best_ref):
    sv = sv_ref[...]
    si = si_ref[...]
    scm = jnp.max(sv, axis=2)
    sci = jnp.min(jnp.where(sv == scm[:, :, None], si, V), axis=2)
    best_ref[...] = jnp.where(scm > tv_ref[...], sci, ti_ref[...])


def _merge(tv, ti, sv, si):
    return pl.pallas_call(
        _merge_body,
        out_shape=jax.ShapeDtypeStruct((B, J), jnp.int32),
    )(tv, ti, sv, si)


@jax.jit
def kernel(action, explore_rate):
    lg_sc, sd_sc, sv, si = _sc_tail(action, explore_rate)
    lg_tc, sd_tc, tv, ti = _tc_head(action, explore_rate)
    best2d = _merge(tv, ti, sv, si)
    logits = jnp.concatenate([lg_tc, lg_sc], axis=1)
    stddev = jnp.concatenate([sd_tc, sd_sc], axis=1)
    return logits, stddev, best2d[:, None, :]


# speedup vs baseline: 1.1438x; 1.1415x over previous
import functools

import jax
import jax.numpy as jnp
from jax import lax
from jax.experimental import pallas as pl
from jax.experimental.pallas import tpu as pltpu
from jax.experimental.pallas import tpu_sc as plsc

B, J, V = 32, 8, 32768
SPLIT = 24576
SC_COLS = V - SPLIT
VC = 2048
NCHUNK_TC = SPLIT // VC
CH = 2048
NCH_SC = SC_COLS // CH
NGRP = CH // 16
NC, NS = 2, 16


def _tc_body(a_ref, e_ref, logits_ref, stddev_ref, bestv_ref, besti_ref,
             macc_ref, cidx_ref):
    j = pl.program_id(0)

    a = a_ref[...]
    e = e_ref[...]

    m = jnp.min(a, axis=1)
    iota_j = jax.lax.broadcasted_iota(jnp.int32, (B, J, VC), 1)
    jsel = jnp.min(jnp.where(a == m[:, None, :], iota_j, J), axis=1)
    s = jnp.sum(jnp.where(iota_j == jsel[:, None, :], e, 0.0), axis=1)
    logits_ref[...] = m
    stddev_ref[...] = s

    @pl.when(j == 0)
    def _():
        macc_ref[...] = a
        cidx_ref[...] = jnp.zeros((B, J, VC), jnp.int32)

    @pl.when(j > 0)
    def _():
        upd = a > macc_ref[...]
        macc_ref[...] = jnp.where(upd, a, macc_ref[...])
        cidx_ref[...] = jnp.where(upd, j, cidx_ref[...])

    @pl.when(j == NCHUNK_TC - 1)
    def _():
        macc = macc_ref[...]
        cm = jnp.max(macc, axis=2)
        lane = jax.lax.broadcasted_iota(jnp.int32, (B, J, VC), 2)
        gidx = cidx_ref[...] * VC + lane
        bestv_ref[...] = cm
        besti_ref[...] = jnp.min(
            jnp.where(macc == cm[:, :, None], gidx, V), axis=2)


def _tc_head(action, explore_rate):
    return pl.pallas_call(
        _tc_body,
        grid=(NCHUNK_TC,),
        in_specs=[
            pl.BlockSpec((B, J, VC), lambda j: (0, 0, j)),
            pl.BlockSpec((B, J, VC), lambda j: (0, 0, j)),
        ],
        out_specs=[
            pl.BlockSpec((B, VC), lambda j: (0, j)),
            pl.BlockSpec((B, VC), lambda j: (0, j)),
            pl.BlockSpec((B, J), lambda j: (0, 0)),
            pl.BlockSpec((B, J), lambda j: (0, 0)),
        ],
        out_shape=[
            jax.ShapeDtypeStruct((B, SPLIT), jnp.float32),
            jax.ShapeDtypeStruct((B, SPLIT), jnp.float32),
            jax.ShapeDtypeStruct((B, J), jnp.float32),
            jax.ShapeDtypeStruct((B, J), jnp.int32),
        ],
        scratch_shapes=[
            pltpu.VMEM((B, J, VC), jnp.float32),
            pltpu.VMEM((B, J, VC), jnp.int32),
        ],
        compiler_params=pltpu.CompilerParams(
            dimension_semantics=("arbitrary",),
        ),
    )(action, explore_rate)


def _sc_tail(action, explore_rate):
    mesh = plsc.VectorSubcoreMesh(core_axis_name="c", subcore_axis_name="s")

    @functools.partial(
        pl.kernel,
        out_type=[
            jax.ShapeDtypeStruct((B, SC_COLS), jnp.float32),
            jax.ShapeDtypeStruct((B, SC_COLS), jnp.float32),
            jax.ShapeDtypeStruct((B, J, 16), jnp.float32),
            jax.ShapeDtypeStruct((B, J, 16), jnp.int32),
        ],
        mesh=mesh,
        scratch_types=[
            pltpu.VMEM((J, CH), jnp.float32),
            pltpu.VMEM((J, CH), jnp.float32),
            pltpu.VMEM((CH,), jnp.float32),
            pltpu.VMEM((CH,), jnp.float32),
            pltpu.VMEM((J, 16), jnp.float32),
            pltpu.VMEM((J, 16), jnp.int32),
        ],
    )
    def sc_kernel(a_hbm, e_hbm, logits_hbm, stddev_hbm, mv_hbm, mi_hbm,
                  a_v, e_v, m_v, s_v, rm_v, ri_v):
        b = lax.axis_index("s") * NC + lax.axis_index("c")
        lane = jax.lax.broadcasted_iota(jnp.int32, (16,), 0)

        for jj in range(J):
            rm_v[jj, :] = jnp.full((16,), -jnp.inf, jnp.float32)
            ri_v[jj, :] = jnp.zeros((16,), jnp.int32)

        @pl.loop(0, NCH_SC)
        def _(c):
            off = SPLIT + c * CH
            pltpu.sync_copy(a_hbm.at[b, :, pl.ds(off, CH)], a_v)
            pltpu.sync_copy(e_hbm.at[b, :, pl.ds(off, CH)], e_v)

            rm0 = [rm_v[jj, :] for jj in range(J)]
            ri0 = [ri_v[jj, :] for jj in range(J)]

            def group(g, carry):
                rm, ri = carry
                col = g * 16
                idx0 = off + col + lane
                a0 = a_v[0, pl.ds(col, 16)]
                m = a0
                s = e_v[0, pl.ds(col, 16)]
                u0 = a0 > rm[0]
                new_rm = [jnp.where(u0, a0, rm[0])]
                new_ri = [jnp.where(u0, idx0, ri[0])]
                for jj in range(1, J):
                    aj = a_v[jj, pl.ds(col, 16)]
                    upd = aj < m
                    m = jnp.where(upd, aj, m)
                    s = jnp.where(upd, e_v[jj, pl.ds(col, 16)], s)
                    uj = aj > rm[jj]
                    new_rm.append(jnp.where(uj, aj, rm[jj]))
                    new_ri.append(jnp.where(uj, idx0, ri[jj]))
                m_v[pl.ds(col, 16)] = m
                s_v[pl.ds(col, 16)] = s
                return new_rm, new_ri

            rm, ri = lax.fori_loop(0, NGRP, group, (rm0, ri0))
            for jj in range(J):
                rm_v[jj, :] = rm[jj]
                ri_v[jj, :] = ri[jj]

            pltpu.sync_copy(m_v, logits_hbm.at[b, pl.ds(c * CH, CH)])
            pltpu.sync_copy(s_v, stddev_hbm.at[b, pl.ds(c * CH, CH)])

        pltpu.sync_copy(rm_v, mv_hbm.at[b])
        pltpu.sync_copy(ri_v, mi_hbm.at[b])

    return sc_kernel(action, explore_rate)


def _merge_body(tv_ref, ti_ref, sv_ref, si_ref, best_ref):
    sv = sv_ref[...]
    si = si_ref[...]
    scm = jnp.max(sv, axis=2)
    sci = jnp.min(jnp.where(sv == scm[:, :, None], si, V), axis=2)
    best_ref[...] = jnp.where(scm > tv_ref[...], sci, ti_ref[...])


def _merge(tv, ti, sv, si):
    return pl.pallas_call(
        _merge_body,
        out_shape=jax.ShapeDtypeStruct((B, J), jnp.int32),
    )(tv, ti, sv, si)


@jax.jit
def kernel(action, explore_rate):
    lg_sc, sd_sc, sv, si = _sc_tail(action, explore_rate)
    lg_tc, sd_tc, tv, ti = _tc_head(action, explore_rate)
    best2d = _merge(tv, ti, sv, si)
    logits = jnp.concatenate([lg_tc, lg_sc], axis=1)
    stddev = jnp.concatenate([sd_tc, sd_sc], axis=1)
    return logits, stddev, best2d[:, None, :]


# speedup vs baseline: 1.4681x; 1.2835x over previous
import jax
import jax.numpy as jnp
from jax.experimental import pallas as pl
from jax.experimental.pallas import tpu as pltpu

B, J, V = 32, 8, 32768
VC = 2048
NCHUNK = V // VC


def _stream_body(a_ref, e_ref, logits_ref, stddev_ref, macc_ref, cidx_ref):
    j = pl.program_id(0)

    a = a_ref[...]
    e = e_ref[...]

    m = jnp.min(a, axis=1)
    iota_j = jax.lax.broadcasted_iota(jnp.int32, (B, J, VC), 1)
    jsel = jnp.min(jnp.where(a == m[:, None, :], iota_j, J), axis=1)
    s = jnp.sum(jnp.where(iota_j == jsel[:, None, :], e, 0.0), axis=1)
    logits_ref[...] = m
    stddev_ref[...] = s

    upd = (a > macc_ref[...]) | (j == 0)
    macc_ref[...] = jnp.where(upd, a, macc_ref[...])
    cidx_ref[...] = jnp.where(upd, j, cidx_ref[...])


def _resolve_body(macc_ref, cidx_ref, best_ref):
    macc = macc_ref[...]
    cm = jnp.max(macc, axis=2)
    lane = jax.lax.broadcasted_iota(jnp.int32, (B, J, VC), 2)
    gidx = cidx_ref[...] * VC + lane
    best_ref[...] = jnp.min(
        jnp.where(macc == cm[:, :, None], gidx, V), axis=2)


@jax.jit
def kernel(action, explore_rate):
    logits, stddev, macc, cidx = pl.pallas_call(
        _stream_body,
        grid=(NCHUNK,),
        in_specs=[
            pl.BlockSpec((B, J, VC), lambda j: (0, 0, j)),
            pl.BlockSpec((B, J, VC), lambda j: (0, 0, j)),
        ],
        out_specs=[
            pl.BlockSpec((B, VC), lambda j: (0, j)),
            pl.BlockSpec((B, VC), lambda j: (0, j)),
            pl.BlockSpec((B, J, VC), lambda j: (0, 0, 0)),
            pl.BlockSpec((B, J, VC), lambda j: (0, 0, 0)),
        ],
        out_shape=[
            jax.ShapeDtypeStruct((B, V), jnp.float32),
            jax.ShapeDtypeStruct((B, V), jnp.float32),
            jax.ShapeDtypeStruct((B, J, VC), jnp.float32),
            jax.ShapeDtypeStruct((B, J, VC), jnp.int32),
        ],
        compiler_params=pltpu.CompilerParams(
            dimension_semantics=("arbitrary",),
        ),
    )(action, explore_rate)

    best2d = pl.pallas_call(
        _resolve_body,
        out_shape=jax.ShapeDtypeStruct((B, J), jnp.int32),
    )(macc, cidx)
    return logits, stddev, best2d[:, None, :]


# speedup vs baseline: 1.5553x; 1.0594x over previous
import jax
import jax.numpy as jnp
from jax.experimental import pallas as pl
from jax.experimental.pallas import tpu as pltpu

B, J, V = 32, 8, 32768
VC = 2048
NCHUNK = V // VC


def _fused_body(a_ref, e_ref, logits_ref, stddev_ref, best_ref,
                macc_ref, cidx_ref):
    j = pl.program_id(0)

    a = a_ref[...]
    e = e_ref[...]

    m = jnp.min(a, axis=1)
    iota_j = jax.lax.broadcasted_iota(jnp.int32, (B, J, VC), 1)
    jsel = jnp.min(jnp.where(a == m[:, None, :], iota_j, J), axis=1)
    s = jnp.sum(jnp.where(iota_j == jsel[:, None, :], e, 0.0), axis=1)
    logits_ref[...] = m
    stddev_ref[...] = s

    upd = (a > macc_ref[...]) | (j == 0)
    macc_ref[...] = jnp.where(upd, a, macc_ref[...])
    cidx_ref[...] = jnp.where(upd, j, cidx_ref[...])

    @pl.when(j == NCHUNK - 1)
    def _():
        macc = macc_ref[...]
        cm = jnp.max(macc, axis=2)
        lane = jax.lax.broadcasted_iota(jnp.int32, (B, J, VC), 2)
        gidx = cidx_ref[...] * VC + lane
        best_ref[...] = jnp.min(
            jnp.where(macc == cm[:, :, None], gidx, V), axis=2)


@jax.jit
def kernel(action, explore_rate):
    logits, stddev, best2d = pl.pallas_call(
        _fused_body,
        grid=(NCHUNK,),
        in_specs=[
            pl.BlockSpec((B, J, VC), lambda j: (0, 0, j)),
            pl.BlockSpec((B, J, VC), lambda j: (0, 0, j)),
        ],
        out_specs=[
            pl.BlockSpec((B, VC), lambda j: (0, j)),
            pl.BlockSpec((B, VC), lambda j: (0, j)),
            pl.BlockSpec((B, J), lambda j: (0, 0)),
        ],
        out_shape=[
            jax.ShapeDtypeStruct((B, V), jnp.float32),
            jax.ShapeDtypeStruct((B, V), jnp.float32),
            jax.ShapeDtypeStruct((B, J), jnp.int32),
        ],
        scratch_shapes=[
            pltpu.VMEM((B, J, VC), jnp.float32),
            pltpu.VMEM((B, J, VC), jnp.int32),
        ],
        compiler_params=pltpu.CompilerParams(
            dimension_semantics=("arbitrary",),
        ),
    )(action, explore_rate)
    return logits, stddev, best2d[:, None, :]
